# baseline (speedup 1.0000x reference)
import jax
import jax.numpy as jnp
from jax import lax
from jax.experimental import pallas as pl
from jax.experimental.pallas import tpu as pltpu
from jax.experimental.pallas import tpu_sc as plsc

B = 4096
NV = 1000
NVP = 1024
WB = NVP // 32
ACT = 8
NW = 32
RPW = B // NW
G = RPW // 16

_C55 = 0x55555555
_C33 = 0x33333333
_C0F = 0x0F0F0F0F
_REP = 0x01010101


def _body(words_hbm, u_hbm, uf_hbm, ia_hbm, fa_hbm, words_v, cum_v, u_v, uf_v, ia_v, fa_v):
    wid = lax.axis_index("s") * 2 + lax.axis_index("c")
    base = wid * RPW

    pltpu.sync_copy(words_hbm.at[pl.ds(base * WB, RPW * WB)], words_v)
    pltpu.sync_copy(u_hbm.at[pl.ds(base, RPW)], u_v)
    pltpu.sync_copy(uf_hbm.at[pl.ds(base * ACT, RPW * ACT)], uf_v)

    iota = lax.iota(jnp.int32, 16)
    c55 = jnp.full((16,), _C55, jnp.int32)
    c33 = jnp.full((16,), _C33, jnp.int32)
    c0f = jnp.full((16,), _C0F, jnp.int32)
    rep = jnp.full((16,), _REP, jnp.int32)
    zero = jnp.zeros((16,), jnp.int32)
    ones = jnp.full((16,), 1, jnp.int32)

    def pc32(v):
        x = v - (lax.shift_right_logical(v, 1) & c55)
        x = (x & c33) + (lax.shift_right_logical(x, 2) & c33)
        x = (x + lax.shift_right_logical(x, 4)) & c0f
        return lax.shift_right_logical(x * rep, 24)

    for g in range(G):
        rows = (iota + (g * 16)) * WB

        def p1(w, carry):
            idx, cum = carry
            for _ in range(4):
                v = plsc.load_gather(words_v, [idx])
                cum = cum + pc32(v)
                plsc.store_scatter(cum_v, [idx], cum)
                idx = idx + ones
            return (idx, cum)

        _, tot = lax.fori_loop(0, WB // 4, p1, (rows, zero))
        u = u_v[pl.ds(g * 16, 16)]
        action = (u * tot.astype(jnp.float32)).astype(jnp.int32)

        lo = zero
        for s in (16, 8, 4, 2, 1):
            svec = jnp.full((16,), s, jnp.int32)
            cand = lo + svec
            c = plsc.load_gather(cum_v, [rows + cand - ones])
            lo = jnp.where(c <= action, cand, lo)
        n = lo
        rbv = plsc.load_gather(cum_v, [rows + jnp.maximum(n - ones, zero)])
        rb = jnp.where(n > zero, rbv, zero)

        vstar = plsc.load_gather(words_v, [rows + n])
        rem = action - rb
        pos = zero
        for level in (16, 8, 4, 2, 1):
            lmask = jnp.full((16,), (1 << level) - 1, jnp.int32)
            lvec = jnp.full((16,), level, jnp.int32)
            half = lax.shift_right_logical(vstar, pos) & lmask
            c = pc32(half)
            go = rem >= c
            pos = pos + jnp.where(go, lvec, zero)
            rem = rem - jnp.where(go, c, zero)

        ia_v[pl.ds(g * 16, 16)] = n * 32 + pos

    def pf(k, _):
        for j in range(4):
            i = k * 4 + j
            x = uf_v[pl.ds(i * 16, 16)]
            fa_v[pl.ds(i * 16, 16)] = x * 2.0 - 1.0
        return 0

    lax.fori_loop(0, RPW * ACT // 64, pf, 0)

    pltpu.sync_copy(ia_v, ia_hbm.at[pl.ds(base, RPW)])
    pltpu.sync_copy(fa_v, fa_hbm.at[pl.ds(base * ACT, RPW * ACT)])


_sc_call = pl.kernel(
    _body,
    out_type=(
        jax.ShapeDtypeStruct((B,), jnp.int32),
        jax.ShapeDtypeStruct((B * ACT,), jnp.float32),
    ),
    mesh=plsc.VectorSubcoreMesh(core_axis_name="c", subcore_axis_name="s"),
    compiler_params=pltpu.CompilerParams(needs_layout_passes=False),
    scratch_types=[
        pltpu.VMEM((RPW * WB,), jnp.int32),
        pltpu.VMEM((RPW * WB,), jnp.int32),
        pltpu.VMEM((RPW,), jnp.float32),
        pltpu.VMEM((RPW * ACT,), jnp.float32),
        pltpu.VMEM((RPW,), jnp.int32),
        pltpu.VMEM((RPW * ACT,), jnp.float32),
    ],
)


@jax.jit
def kernel(states, mask, u_int, u_float):
    del states
    m3 = jnp.pad(mask, ((0, 0), (0, NVP - NV))).reshape(B, WB, 32)
    weights = jnp.left_shift(jnp.int32(1), jnp.arange(32, dtype=jnp.int32))
    words = jnp.sum(m3 * weights, axis=-1, dtype=jnp.int32).reshape(B * WB)
    ia, fa = _sc_call(words, u_int, u_float.reshape(B * ACT))
    return ia, fa.reshape(B, ACT)


# speedup vs baseline: 1.2055x; 1.2055x over previous
import jax
import jax.numpy as jnp
from jax import lax
from jax.experimental import pallas as pl
from jax.experimental.pallas import tpu as pltpu
from jax.experimental.pallas import tpu_sc as plsc

B = 4096
NV = 1000
NVP = 1024
WB = NVP // 32
ACT = 8
NW = 32
RPW = B // NW
G = RPW // 16

_C55 = 0x55555555
_C33 = 0x33333333
_C0F = 0x0F0F0F0F
_REP = 0x01010101


def _body(words_hbm, u_hbm, ia_hbm, words_v, cum_v, u_v, ia_v, sem_w, sem_u):
    wid = lax.axis_index("s") * 2 + lax.axis_index("c")
    base = wid * RPW

    cp_w = pltpu.async_copy(words_hbm.at[pl.ds(base * WB, RPW * WB)], words_v, sem_w)
    cp_u = pltpu.async_copy(u_hbm.at[pl.ds(base, RPW)], u_v, sem_u)

    iota = lax.iota(jnp.int32, 16)
    c55 = jnp.full((16,), _C55, jnp.int32)
    c33 = jnp.full((16,), _C33, jnp.int32)
    c0f = jnp.full((16,), _C0F, jnp.int32)
    rep = jnp.full((16,), _REP, jnp.int32)
    zero = jnp.zeros((16,), jnp.int32)
    ones = jnp.full((16,), 1, jnp.int32)

    def pc32(v):
        x = v - (lax.shift_right_logical(v, 1) & c55)
        x = (x & c33) + (lax.shift_right_logical(x, 2) & c33)
        x = (x + lax.shift_right_logical(x, 4)) & c0f
        return lax.shift_right_logical(x * rep, 24)

    cp_w.wait()
    cp_u.wait()

    for g in range(G):
        rows = (iota + (g * 16)) * WB

        idx = rows
        cum = zero
        for _ in range(WB):
            v = plsc.load_gather(words_v, [idx])
            cum = cum + pc32(v)
            plsc.store_scatter(cum_v, [idx], cum)
            idx = idx + ones
        tot = cum
        u = u_v[pl.ds(g * 16, 16)]
        action = (u * tot.astype(jnp.float32)).astype(jnp.int32)

        lo = zero
        for s in (16, 8, 4, 2, 1):
            svec = jnp.full((16,), s, jnp.int32)
            cand = lo + svec
            c = plsc.load_gather(cum_v, [rows + cand - ones])
            lo = jnp.where(c <= action, cand, lo)
        n = lo
        rbv = plsc.load_gather(cum_v, [rows + jnp.maximum(n - ones, zero)])
        rb = jnp.where(n > zero, rbv, zero)

        vstar = plsc.load_gather(words_v, [rows + n])
        rem = action - rb
        pos = zero
        for level in (16, 8, 4, 2, 1):
            lmask = jnp.full((16,), (1 << level) - 1, jnp.int32)
            lvec = jnp.full((16,), level, jnp.int32)
            half = lax.shift_right_logical(vstar, pos) & lmask
            c = pc32(half)
            go = rem >= c
            pos = pos + jnp.where(go, lvec, zero)
            rem = rem - jnp.where(go, c, zero)

        ia_v[pl.ds(g * 16, 16)] = n * 32 + pos

    pltpu.sync_copy(ia_v, ia_hbm.at[pl.ds(base, RPW)])


_sc_call = pl.kernel(
    _body,
    out_type=jax.ShapeDtypeStruct((B,), jnp.int32),
    mesh=plsc.VectorSubcoreMesh(core_axis_name="c", subcore_axis_name="s"),
    compiler_params=pltpu.CompilerParams(needs_layout_passes=False),
    scratch_types=[
        pltpu.VMEM((RPW * WB,), jnp.int32),
        pltpu.VMEM((RPW * WB,), jnp.int32),
        pltpu.VMEM((RPW,), jnp.float32),
        pltpu.VMEM((RPW,), jnp.int32),
        pltpu.SemaphoreType.DMA,
        pltpu.SemaphoreType.DMA,
    ],
)


@jax.jit
def kernel(states, mask, u_int, u_float):
    del states
    m3 = jnp.pad(mask, ((0, 0), (0, NVP - NV))).reshape(B, WB, 32)
    weights = jnp.left_shift(jnp.int32(1), jnp.arange(32, dtype=jnp.int32))
    words = jnp.sum(m3 * weights, axis=-1, dtype=jnp.int32).reshape(B * WB)
    ia = _sc_call(words, u_int)
    return ia, u_float * 2.0 - 1.0
